# Initial kernel scaffold; baseline (speedup 1.0000x reference)
import jax
import jax.numpy as jnp
from jax.experimental import pallas as pl


def kernel(x, table, W_ih, W_hh, b_ih, b_hh, W_fc, b_fc):
    raise NotImplementedError("write your pallas kernel here")


# speedup vs baseline: 6.6746x; 6.6746x over previous
import functools

import jax
import jax.numpy as jnp
from jax import lax
from jax.experimental import pallas as pl
from jax.experimental.pallas import tpu as pltpu
from jax.experimental.pallas import tpu_sc as plsc

_NUM_CORES = 2
_NUM_SUBCORES = 16
_NW = _NUM_CORES * _NUM_SUBCORES
_IDXW = 128
_PACK = 4


def _make_sc_gather(n_idx, e):
    rows_per_w = n_idx // _NW
    kj = rows_per_w // _IDXW
    mesh = plsc.VectorSubcoreMesh(core_axis_name="c", subcore_axis_name="s")

    @functools.partial(
        pl.kernel,
        mesh=mesh,
        out_type=jax.ShapeDtypeStruct((n_idx, e), jnp.float32),
        scratch_types=[
            pltpu.VMEM((kj, _IDXW), jnp.int32),
            pltpu.VMEM((_IDXW, e), jnp.float32),
            pltpu.SemaphoreType.DMA,
        ],
    )
    def sc_gather(idx_hbm, table_hbm, out_hbm, idx_v, rows_v, sem):
        wid = lax.axis_index("s") * _NUM_CORES + lax.axis_index("c")
        pltpu.sync_copy(idx_hbm.at[wid], idx_v)
        base = wid * rows_per_w

        def body(j, carry):
            pltpu.async_copy(table_hbm.at[idx_v.at[j]], rows_v, sem).wait()
            pltpu.sync_copy(rows_v, out_hbm.at[pl.ds(base + j * _IDXW, _IDXW)])
            return carry

        lax.fori_loop(0, kj, body, 0)

    return sc_gather


def _lstm_step_kernel(emb_ref, sel_ref, wih_ref, whh_ref, bias_ref, wfc_ref,
                      bfc_ref, out_ref, h_ref, c_ref):
    t = pl.program_id(0)
    nt = pl.num_programs(0)
    h_dim = whh_ref.shape[0]
    lanes = emb_ref.shape[1]

    @pl.when(t == 0)
    def _():
        h_ref[...] = jnp.zeros_like(h_ref)
        c_ref[...] = jnp.zeros_like(c_ref)

    grp = lax.broadcasted_iota(jnp.int32, emb_ref.shape, 1) // (lanes // _PACK)
    x_t = jnp.where(grp == sel_ref[...], emb_ref[...], 0.0)
    h = h_ref[...]
    gates = (jnp.dot(x_t, wih_ref[...], preferred_element_type=jnp.float32)
             + jnp.dot(h, whh_ref[...], preferred_element_type=jnp.float32)
             + bias_ref[...])
    i_g = jax.nn.sigmoid(gates[:, 0:h_dim])
    f_g = jax.nn.sigmoid(gates[:, h_dim:2 * h_dim])
    g_g = jnp.tanh(gates[:, 2 * h_dim:3 * h_dim])
    o_g = jax.nn.sigmoid(gates[:, 3 * h_dim:4 * h_dim])
    c_new = f_g * c_ref[...] + i_g * g_g
    h_new = o_g * jnp.tanh(c_new)
    h_ref[...] = h_new
    c_ref[...] = c_new

    @pl.when(t == nt - 1)
    def _():
        out_ref[...] = (jnp.dot(h_new, wfc_ref[...],
                                preferred_element_type=jnp.float32)
                        + bfc_ref[...])


def _lstm_fc(emb_flat, sel, wih_t, whh_t, bias, wfc_t, bfc, b, t_steps):
    e = emb_flat.shape[1]
    h4 = wih_t.shape[1]
    h_dim = h4 // 4
    out_dim = wfc_t.shape[1]
    return pl.pallas_call(
        _lstm_step_kernel,
        grid=(t_steps,),
        in_specs=[
            pl.BlockSpec((b, e), lambda t: (t, 0)),
            pl.BlockSpec((b, 1), lambda t: (t, 0)),
            pl.BlockSpec((e, h4), lambda t: (0, 0)),
            pl.BlockSpec((h_dim, h4), lambda t: (0, 0)),
            pl.BlockSpec((1, h4), lambda t: (0, 0)),
            pl.BlockSpec((h_dim, out_dim), lambda t: (0, 0)),
            pl.BlockSpec((1, out_dim), lambda t: (0, 0)),
        ],
        out_specs=pl.BlockSpec((b, out_dim), lambda t: (0, 0)),
        out_shape=jax.ShapeDtypeStruct((b, out_dim), jnp.float32),
        scratch_shapes=[
            pltpu.VMEM((b, h_dim), jnp.float32),
            pltpu.VMEM((b, h_dim), jnp.float32),
        ],
        compiler_params=pltpu.CompilerParams(
            dimension_semantics=("arbitrary",),
        ),
    )(emb_flat, sel, wih_t, whh_t, bias, wfc_t, bfc)


def kernel(x, table, W_ih, W_hh, b_ih, b_hh, W_fc, b_fc):
    b, t_steps = x.shape
    e = table.shape[1]
    h4 = W_ih.shape[0]
    out_dim = W_fc.shape[0]

    v = table.shape[0]
    tbl_packed = table.reshape(v // _PACK, e * _PACK)
    xt = x.astype(jnp.int32).T
    idx3d = (xt // _PACK).reshape(_NW, -1, _IDXW)
    sel = (xt % _PACK).reshape(-1, 1)
    emb_flat = _make_sc_gather(b * t_steps, e * _PACK)(idx3d, tbl_packed)

    wih_t = jnp.tile(W_ih.T, (_PACK, 1))
    whh_t = W_hh.T
    bias = (b_ih + b_hh).reshape(1, h4)
    wfc_t = W_fc.T
    bfc = b_fc.reshape(1, out_dim)
    return _lstm_fc(emb_flat, sel, wih_t, whh_t, bias, wfc_t, bfc, b, t_steps)


# speedup vs baseline: 10.6428x; 1.5945x over previous
import functools

import jax
import jax.numpy as jnp
from jax import lax
from jax.experimental import pallas as pl
from jax.experimental.pallas import tpu as pltpu
from jax.experimental.pallas import tpu_sc as plsc

_NUM_CORES = 2
_NUM_SUBCORES = 16
_NW = _NUM_CORES * _NUM_SUBCORES
_IDXW = 128


def _make_sc_gather(n_idx, e):
    rows_per_w = n_idx // _NW
    kj = rows_per_w // _IDXW
    mesh = plsc.VectorSubcoreMesh(core_axis_name="c", subcore_axis_name="s")

    @functools.partial(
        pl.kernel,
        mesh=mesh,
        out_type=jax.ShapeDtypeStruct((n_idx, e), jnp.float32),
        scratch_types=[
            pltpu.VMEM((kj, _IDXW), jnp.int32),
            pltpu.VMEM((_IDXW, e), jnp.float32),
            pltpu.VMEM((_IDXW, e), jnp.float32),
            pltpu.SemaphoreType.DMA,
            pltpu.SemaphoreType.DMA,
        ],
    )
    def sc_gather(idx_hbm, table_hbm, out_hbm, idx_v, rows_a, rows_b, sem_a,
                  sem_b):
        wid = lax.axis_index("s") * _NUM_CORES + lax.axis_index("c")
        pltpu.sync_copy(idx_hbm.at[wid], idx_v)
        base = wid * rows_per_w

        bufs = (rows_a, rows_b)
        sems = (sem_a, sem_b)
        pltpu.async_copy(table_hbm.at[idx_v.at[0]], rows_a, sem_a)

        def body(jj, carry):
            for p in range(2):
                j = jj * 2 + p
                cur, nxt = bufs[p], bufs[1 - p]
                scur, snxt = sems[p], sems[1 - p]

                @pl.when(j + 1 < kj)
                def _():
                    pltpu.async_copy(table_hbm.at[idx_v.at[j + 1]], nxt, snxt)

                pltpu.make_async_copy(table_hbm.at[idx_v.at[j]], cur,
                                      scur).wait()
                pltpu.sync_copy(cur, out_hbm.at[pl.ds(base + j * _IDXW,
                                                      _IDXW)])
            return carry

        lax.fori_loop(0, kj // 2, body, 0)
        if kj % 2 == 1:
            j = kj - 1
            pltpu.make_async_copy(table_hbm.at[idx_v.at[j]], bufs[j % 2],
                                  sems[j % 2]).wait()
            pltpu.sync_copy(bufs[j % 2],
                            out_hbm.at[pl.ds(base + j * _IDXW, _IDXW)])

    return sc_gather


_PACK = 4
_RC = 8192


def _repack_kernel(tt_ref, emap_ref, out_ref, *, vtotal):
    e = tt_ref.shape[0]
    lanes = _PACK * e
    sub = tt_ref.shape[1] // _PACK
    valid = vtotal - pl.program_id(0) * _RC
    col = lax.broadcasted_iota(jnp.int32, tt_ref.shape, 1)
    tt = jnp.where(col < valid, tt_ref[...], 0.0)
    acc = None
    for p in range(_PACK):
        d = lax.dot_general(
            tt[:, p * sub:(p + 1) * sub],
            emap_ref[:, p * lanes:(p + 1) * lanes],
            (((0,), (0,)), ((), ())),
            preferred_element_type=jnp.float32)
        acc = d if acc is None else acc + d
    out_ref[...] = acc


def _repack(table_t):
    e, v = table_t.shape
    lanes = _PACK * e
    sub = _RC // _PACK
    nblk = (v + _RC - 1) // _RC
    eye = jnp.eye(e, dtype=jnp.float32)
    emap = jnp.concatenate(
        [jnp.pad(eye, ((0, 0), (p * e, lanes - (p + 1) * e)))
         for p in range(_PACK)], axis=1)
    return pl.pallas_call(
        functools.partial(_repack_kernel, vtotal=v),
        grid=(nblk,),
        in_specs=[
            pl.BlockSpec((e, _RC), lambda k: (0, k)),
            pl.BlockSpec((e, _PACK * lanes), lambda k: (0, 0)),
        ],
        out_specs=pl.BlockSpec((sub, lanes), lambda k: (k, 0)),
        out_shape=jax.ShapeDtypeStruct((nblk * sub, lanes), jnp.float32),
        compiler_params=pltpu.CompilerParams(
            fuse_transposed_lhs_in_matmul=True,
        ),
    )(table_t, emap)


def _cell(x_t, h, c, wih_ref, whh_ref, bias_ref, h_dim):
    gates = (jnp.dot(x_t, wih_ref[...], preferred_element_type=jnp.float32)
             + jnp.dot(h, whh_ref[...], preferred_element_type=jnp.float32)
             + bias_ref[...])
    i_g = jax.nn.sigmoid(gates[:, 0:h_dim])
    f_g = jax.nn.sigmoid(gates[:, h_dim:2 * h_dim])
    g_g = jnp.tanh(gates[:, 2 * h_dim:3 * h_dim])
    o_g = jax.nn.sigmoid(gates[:, 3 * h_dim:4 * h_dim])
    c_new = f_g * c + i_g * g_g
    h_new = o_g * jnp.tanh(c_new)
    return h_new, c_new


def _x_sel(emb_ref, sel_ref):
    lanes = emb_ref.shape[1]
    grp = lax.broadcasted_iota(jnp.int32, emb_ref.shape, 1) // (lanes // _PACK)
    return jnp.where(grp == sel_ref[...], emb_ref[...], 0.0)


def _lstm_first_kernel(emb_ref, sel_ref, wih_ref, whh_ref, bias_ref,
                       ho_ref, co_ref, h_ref, c_ref):
    t = pl.program_id(0)
    nt = pl.num_programs(0)
    h_dim = whh_ref.shape[0]

    @pl.when(t == 0)
    def _():
        h_ref[...] = jnp.zeros_like(h_ref)
        c_ref[...] = jnp.zeros_like(c_ref)

    x_t = _x_sel(emb_ref, sel_ref)
    h_new, c_new = _cell(x_t, h_ref[...], c_ref[...], wih_ref, whh_ref,
                         bias_ref, h_dim)
    h_ref[...] = h_new
    c_ref[...] = c_new

    @pl.when(t == nt - 1)
    def _():
        ho_ref[...] = h_new
        co_ref[...] = c_new


def _lstm_final_kernel(emb_ref, sel_ref, wih_ref, whh_ref, bias_ref,
                       h0_ref, c0_ref, wfc_ref, bfc_ref, out_ref,
                       h_ref, c_ref):
    t = pl.program_id(0)
    nt = pl.num_programs(0)
    h_dim = whh_ref.shape[0]

    @pl.when(t == 0)
    def _():
        h_ref[...] = h0_ref[...]
        c_ref[...] = c0_ref[...]

    x_t = _x_sel(emb_ref, sel_ref)
    h_new, c_new = _cell(x_t, h_ref[...], c_ref[...], wih_ref, whh_ref,
                         bias_ref, h_dim)
    h_ref[...] = h_new
    c_ref[...] = c_new

    @pl.when(t == nt - 1)
    def _():
        out_ref[...] = (jnp.dot(h_new, wfc_ref[...],
                                preferred_element_type=jnp.float32)
                        + bfc_ref[...])


def _lstm_first(emb_flat, sel, wih_t, whh_t, bias, b, t_steps):
    e = emb_flat.shape[1]
    h4 = wih_t.shape[1]
    h_dim = h4 // 4
    return pl.pallas_call(
        _lstm_first_kernel,
        grid=(t_steps,),
        in_specs=[
            pl.BlockSpec((b, e), lambda t: (t, 0)),
            pl.BlockSpec((b, 1), lambda t: (t, 0)),
            pl.BlockSpec((e, h4), lambda t: (0, 0)),
            pl.BlockSpec((h_dim, h4), lambda t: (0, 0)),
            pl.BlockSpec((1, h4), lambda t: (0, 0)),
        ],
        out_specs=[
            pl.BlockSpec((b, h_dim), lambda t: (0, 0)),
            pl.BlockSpec((b, h_dim), lambda t: (0, 0)),
        ],
        out_shape=[
            jax.ShapeDtypeStruct((b, h_dim), jnp.float32),
            jax.ShapeDtypeStruct((b, h_dim), jnp.float32),
        ],
        scratch_shapes=[
            pltpu.VMEM((b, h_dim), jnp.float32),
            pltpu.VMEM((b, h_dim), jnp.float32),
        ],
        compiler_params=pltpu.CompilerParams(
            dimension_semantics=("arbitrary",),
        ),
    )(emb_flat, sel, wih_t, whh_t, bias)


def _lstm_final(emb_flat, sel, wih_t, whh_t, bias, h0, c0, wfc_t, bfc, b,
                t_steps):
    e = emb_flat.shape[1]
    h4 = wih_t.shape[1]
    h_dim = h4 // 4
    out_dim = wfc_t.shape[1]
    return pl.pallas_call(
        _lstm_final_kernel,
        grid=(t_steps,),
        in_specs=[
            pl.BlockSpec((b, e), lambda t: (t, 0)),
            pl.BlockSpec((b, 1), lambda t: (t, 0)),
            pl.BlockSpec((e, h4), lambda t: (0, 0)),
            pl.BlockSpec((h_dim, h4), lambda t: (0, 0)),
            pl.BlockSpec((1, h4), lambda t: (0, 0)),
            pl.BlockSpec((b, h_dim), lambda t: (0, 0)),
            pl.BlockSpec((b, h_dim), lambda t: (0, 0)),
            pl.BlockSpec((h_dim, out_dim), lambda t: (0, 0)),
            pl.BlockSpec((1, out_dim), lambda t: (0, 0)),
        ],
        out_specs=pl.BlockSpec((b, out_dim), lambda t: (0, 0)),
        out_shape=jax.ShapeDtypeStruct((b, out_dim), jnp.float32),
        scratch_shapes=[
            pltpu.VMEM((b, h_dim), jnp.float32),
            pltpu.VMEM((b, h_dim), jnp.float32),
        ],
        compiler_params=pltpu.CompilerParams(
            dimension_semantics=("arbitrary",),
        ),
    )(emb_flat, sel, wih_t, whh_t, bias, h0, c0, wfc_t, bfc)


def kernel(x, table, W_ih, W_hh, b_ih, b_hh, W_fc, b_fc):
    b, t_steps = x.shape
    e = table.shape[1]
    h4 = W_ih.shape[0]
    out_dim = W_fc.shape[0]

    tbl_packed = _repack(table.T)
    xt = x.astype(jnp.int32).T
    sub = _RC // _PACK
    rows = (xt // _RC) * sub + (xt % sub)
    sels = (xt // sub) % _PACK
    th = t_steps // 2
    gat = _make_sc_gather(b * th, e * _PACK)
    idx_a = rows[:th].reshape(_NW, -1, _IDXW)
    idx_b = rows[th:].reshape(_NW, -1, _IDXW)
    emb_a = gat(idx_a, tbl_packed)
    emb_b = gat(idx_b, tbl_packed)
    sel_a = sels[:th].reshape(-1, 1)
    sel_b = sels[th:].reshape(-1, 1)

    wih_t = jnp.tile(W_ih.T, (_PACK, 1))
    whh_t = W_hh.T
    bias = (b_ih + b_hh).reshape(1, h4)
    wfc_t = W_fc.T
    bfc = b_fc.reshape(1, out_dim)
    h0, c0 = _lstm_first(emb_a, sel_a, wih_t, whh_t, bias, b, th)
    return _lstm_final(emb_b, sel_b, wih_t, whh_t, bias, h0, c0, wfc_t, bfc,
                       b, t_steps - th)


# speedup vs baseline: 11.6558x; 1.0952x over previous
import functools

import jax
import jax.numpy as jnp
from jax import lax
from jax.experimental import pallas as pl
from jax.experimental.pallas import tpu as pltpu
from jax.experimental.pallas import tpu_sc as plsc

_NUM_CORES = 2
_NUM_SUBCORES = 16
_NW = _NUM_CORES * _NUM_SUBCORES
_IDXW = 128


def _make_sc_gather(n_idx, e):
    rows_per_w = n_idx // _NW
    kj = rows_per_w // _IDXW
    mesh = plsc.VectorSubcoreMesh(core_axis_name="c", subcore_axis_name="s")

    @functools.partial(
        pl.kernel,
        mesh=mesh,
        out_type=jax.ShapeDtypeStruct((n_idx, e), jnp.float32),
        scratch_types=[
            pltpu.VMEM((kj, _IDXW), jnp.int32),
            pltpu.VMEM((_IDXW, e), jnp.float32),
            pltpu.VMEM((_IDXW, e), jnp.float32),
            pltpu.SemaphoreType.DMA,
            pltpu.SemaphoreType.DMA,
        ],
    )
    def sc_gather(idx_hbm, table_hbm, out_hbm, idx_v, rows_a, rows_b, sem_a,
                  sem_b):
        wid = lax.axis_index("s") * _NUM_CORES + lax.axis_index("c")
        pltpu.sync_copy(idx_hbm.at[wid], idx_v)
        base = wid * rows_per_w

        bufs = (rows_a, rows_b)
        sems = (sem_a, sem_b)
        pltpu.async_copy(table_hbm.at[idx_v.at[0]], rows_a, sem_a)

        def body(jj, carry):
            for p in range(2):
                j = jj * 2 + p
                cur, nxt = bufs[p], bufs[1 - p]
                scur, snxt = sems[p], sems[1 - p]

                @pl.when(j + 1 < kj)
                def _():
                    pltpu.async_copy(table_hbm.at[idx_v.at[j + 1]], nxt, snxt)

                pltpu.make_async_copy(table_hbm.at[idx_v.at[j]], cur,
                                      scur).wait()
                pltpu.sync_copy(cur, out_hbm.at[pl.ds(base + j * _IDXW,
                                                      _IDXW)])
            return carry

        lax.fori_loop(0, kj // 2, body, 0)
        if kj % 2 == 1:
            j = kj - 1
            pltpu.make_async_copy(table_hbm.at[idx_v.at[j]], bufs[j % 2],
                                  sems[j % 2]).wait()
            pltpu.sync_copy(bufs[j % 2],
                            out_hbm.at[pl.ds(base + j * _IDXW, _IDXW)])

    return sc_gather


_PACK = 4
_RC = 8192


def _repack_kernel(tt_ref, emap_ref, out_ref, *, vtotal):
    e = tt_ref.shape[0]
    lanes = _PACK * e
    sub = tt_ref.shape[1] // _PACK
    valid = vtotal - pl.program_id(0) * _RC
    col = lax.broadcasted_iota(jnp.int32, tt_ref.shape, 1)
    tt = jnp.where(col < valid, tt_ref[...], 0.0)
    acc = None
    for p in range(_PACK):
        d = lax.dot_general(
            tt[:, p * sub:(p + 1) * sub],
            emap_ref[:, p * lanes:(p + 1) * lanes],
            (((0,), (0,)), ((), ())),
            preferred_element_type=jnp.float32)
        acc = d if acc is None else acc + d
    out_ref[...] = acc


def _repack(table_t):
    e, v = table_t.shape
    lanes = _PACK * e
    sub = _RC // _PACK
    nblk = (v + _RC - 1) // _RC
    eye = jnp.eye(e, dtype=jnp.float32)
    emap = jnp.concatenate(
        [jnp.pad(eye, ((0, 0), (p * e, lanes - (p + 1) * e)))
         for p in range(_PACK)], axis=1)
    return pl.pallas_call(
        functools.partial(_repack_kernel, vtotal=v),
        grid=(nblk,),
        in_specs=[
            pl.BlockSpec((e, _RC), lambda k: (0, k)),
            pl.BlockSpec((e, _PACK * lanes), lambda k: (0, 0)),
        ],
        out_specs=pl.BlockSpec((sub, lanes), lambda k: (k, 0)),
        out_shape=jax.ShapeDtypeStruct((nblk * sub, lanes), jnp.float32),
        compiler_params=pltpu.CompilerParams(
            fuse_transposed_lhs_in_matmul=True,
        ),
    )(table_t, emap)


def _cell(x_t, h, c, wih_ref, whh_ref, bias_ref, h_dim):
    gates = (jnp.dot(x_t, wih_ref[...], preferred_element_type=jnp.float32)
             + jnp.dot(h, whh_ref[...], preferred_element_type=jnp.float32)
             + bias_ref[...])
    i_g = jax.nn.sigmoid(gates[:, 0:h_dim])
    f_g = jax.nn.sigmoid(gates[:, h_dim:2 * h_dim])
    g_g = jnp.tanh(gates[:, 2 * h_dim:3 * h_dim])
    o_g = jax.nn.sigmoid(gates[:, 3 * h_dim:4 * h_dim])
    c_new = f_g * c + i_g * g_g
    h_new = o_g * jnp.tanh(c_new)
    return h_new, c_new


def _x_sel(emb_ref, sel_ref):
    lanes = emb_ref.shape[1]
    sel_col = sel_ref[0].T
    grp = lax.broadcasted_iota(jnp.int32, emb_ref.shape, 1) // (lanes // _PACK)
    return jnp.where(grp == sel_col, emb_ref[...], 0.0)


def _lstm_first_kernel(emb_ref, sel_ref, wih_ref, whh_ref, bias_ref,
                       ho_ref, co_ref, h_ref, c_ref):
    t = pl.program_id(0)
    nt = pl.num_programs(0)
    h_dim = whh_ref.shape[0]

    @pl.when(t == 0)
    def _():
        h_ref[...] = jnp.zeros_like(h_ref)
        c_ref[...] = jnp.zeros_like(c_ref)

    x_t = _x_sel(emb_ref, sel_ref)
    h_new, c_new = _cell(x_t, h_ref[...], c_ref[...], wih_ref, whh_ref,
                         bias_ref, h_dim)
    h_ref[...] = h_new
    c_ref[...] = c_new

    @pl.when(t == nt - 1)
    def _():
        ho_ref[...] = h_new
        co_ref[...] = c_new


def _lstm_final_kernel(emb_ref, sel_ref, wih_ref, whh_ref, bias_ref,
                       h0_ref, c0_ref, wfc_ref, bfc_ref, out_ref,
                       h_ref, c_ref):
    t = pl.program_id(0)
    nt = pl.num_programs(0)
    h_dim = whh_ref.shape[0]

    @pl.when(t == 0)
    def _():
        h_ref[...] = h0_ref[...]
        c_ref[...] = c0_ref[...]

    x_t = _x_sel(emb_ref, sel_ref)
    h_new, c_new = _cell(x_t, h_ref[...], c_ref[...], wih_ref, whh_ref,
                         bias_ref, h_dim)
    h_ref[...] = h_new
    c_ref[...] = c_new

    @pl.when(t == nt - 1)
    def _():
        out_ref[...] = (jnp.dot(h_new, wfc_ref[...],
                                preferred_element_type=jnp.float32)
                        + bfc_ref[...])


def _lstm_first(emb_flat, sel, wih_t, whh_t, bias, b, t_steps):
    e = emb_flat.shape[1]
    h4 = wih_t.shape[1]
    h_dim = h4 // 4
    return pl.pallas_call(
        _lstm_first_kernel,
        grid=(t_steps,),
        in_specs=[
            pl.BlockSpec((b, e), lambda t: (t, 0)),
            pl.BlockSpec((1, 1, b), lambda t: (t, 0, 0)),
            pl.BlockSpec((e, h4), lambda t: (0, 0)),
            pl.BlockSpec((h_dim, h4), lambda t: (0, 0)),
            pl.BlockSpec((1, h4), lambda t: (0, 0)),
        ],
        out_specs=[
            pl.BlockSpec((b, h_dim), lambda t: (0, 0)),
            pl.BlockSpec((b, h_dim), lambda t: (0, 0)),
        ],
        out_shape=[
            jax.ShapeDtypeStruct((b, h_dim), jnp.float32),
            jax.ShapeDtypeStruct((b, h_dim), jnp.float32),
        ],
        scratch_shapes=[
            pltpu.VMEM((b, h_dim), jnp.float32),
            pltpu.VMEM((b, h_dim), jnp.float32),
        ],
        compiler_params=pltpu.CompilerParams(
            dimension_semantics=("arbitrary",),
        ),
    )(emb_flat, sel, wih_t, whh_t, bias)


def _lstm_final(emb_flat, sel, wih_t, whh_t, bias, h0, c0, wfc_t, bfc, b,
                t_steps):
    e = emb_flat.shape[1]
    h4 = wih_t.shape[1]
    h_dim = h4 // 4
    out_dim = wfc_t.shape[1]
    return pl.pallas_call(
        _lstm_final_kernel,
        grid=(t_steps,),
        in_specs=[
            pl.BlockSpec((b, e), lambda t: (t, 0)),
            pl.BlockSpec((1, 1, b), lambda t: (t, 0, 0)),
            pl.BlockSpec((e, h4), lambda t: (0, 0)),
            pl.BlockSpec((h_dim, h4), lambda t: (0, 0)),
            pl.BlockSpec((1, h4), lambda t: (0, 0)),
            pl.BlockSpec((b, h_dim), lambda t: (0, 0)),
            pl.BlockSpec((b, h_dim), lambda t: (0, 0)),
            pl.BlockSpec((h_dim, out_dim), lambda t: (0, 0)),
            pl.BlockSpec((1, out_dim), lambda t: (0, 0)),
        ],
        out_specs=pl.BlockSpec((b, out_dim), lambda t: (0, 0)),
        out_shape=jax.ShapeDtypeStruct((b, out_dim), jnp.float32),
        scratch_shapes=[
            pltpu.VMEM((b, h_dim), jnp.float32),
            pltpu.VMEM((b, h_dim), jnp.float32),
        ],
        compiler_params=pltpu.CompilerParams(
            dimension_semantics=("arbitrary",),
        ),
    )(emb_flat, sel, wih_t, whh_t, bias, h0, c0, wfc_t, bfc)


def kernel(x, table, W_ih, W_hh, b_ih, b_hh, W_fc, b_fc):
    b, t_steps = x.shape
    e = table.shape[1]
    h4 = W_ih.shape[0]
    out_dim = W_fc.shape[0]

    tbl_packed = _repack(table.T)
    xt = x.astype(jnp.int32).T
    sub = _RC // _PACK
    rows = (xt // _RC) * sub + (xt % sub)
    sels = (xt // sub) % _PACK
    th = t_steps // 2
    gat = _make_sc_gather(b * th, e * _PACK)
    idx_a = rows[:th].reshape(_NW, -1, _IDXW)
    idx_b = rows[th:].reshape(_NW, -1, _IDXW)
    emb_a = gat(idx_a, tbl_packed)
    emb_b = gat(idx_b, tbl_packed)
    sel_a = sels[:th].reshape(th, 1, b)
    sel_b = sels[th:].reshape(t_steps - th, 1, b)

    wih_t = jnp.tile(W_ih.T, (_PACK, 1))
    whh_t = W_hh.T
    bias = (b_ih + b_hh).reshape(1, h4)
    wfc_t = W_fc.T
    bfc = b_fc.reshape(1, out_dim)
    h0, c0 = _lstm_first(emb_a, sel_a, wih_t, whh_t, bias, b, th)
    return _lstm_final(emb_b, sel_b, wih_t, whh_t, bias, h0, c0, wfc_t, bfc,
                       b, t_steps - th)


# speedup vs baseline: 12.8221x; 1.1001x over previous
import functools

import jax
import jax.numpy as jnp
from jax import lax
from jax.experimental import pallas as pl
from jax.experimental.pallas import tpu as pltpu
from jax.experimental.pallas import tpu_sc as plsc

_NUM_CORES = 2
_NUM_SUBCORES = 16
_NW = _NUM_CORES * _NUM_SUBCORES
_IDXW = 128


def _make_sc_gather(n_idx, e):
    rows_per_w = n_idx // _NW
    kj = rows_per_w // _IDXW
    mesh = plsc.VectorSubcoreMesh(core_axis_name="c", subcore_axis_name="s")

    @functools.partial(
        pl.kernel,
        mesh=mesh,
        out_type=jax.ShapeDtypeStruct((n_idx, e), jnp.float32),
        scratch_types=[
            pltpu.VMEM((kj, _IDXW), jnp.int32),
            pltpu.VMEM((_IDXW, e), jnp.float32),
            pltpu.VMEM((_IDXW, e), jnp.float32),
            pltpu.SemaphoreType.DMA,
            pltpu.SemaphoreType.DMA,
        ],
    )
    def sc_gather(idx_hbm, table_hbm, out_hbm, idx_v, rows_a, rows_b, sem_a,
                  sem_b):
        wid = lax.axis_index("s") * _NUM_CORES + lax.axis_index("c")
        pltpu.sync_copy(idx_hbm.at[wid], idx_v)
        base = wid * rows_per_w

        bufs = (rows_a, rows_b)
        sems = (sem_a, sem_b)
        pltpu.async_copy(table_hbm.at[idx_v.at[0]], rows_a, sem_a)

        def body(jj, carry):
            for p in range(2):
                j = jj * 2 + p
                cur, nxt = bufs[p], bufs[1 - p]
                scur, snxt = sems[p], sems[1 - p]

                @pl.when(j + 1 < kj)
                def _():
                    pltpu.async_copy(table_hbm.at[idx_v.at[j + 1]], nxt, snxt)

                pltpu.make_async_copy(table_hbm.at[idx_v.at[j]], cur,
                                      scur).wait()
                pltpu.sync_copy(cur, out_hbm.at[pl.ds(base + j * _IDXW,
                                                      _IDXW)])
            return carry

        lax.fori_loop(0, kj // 2, body, 0)
        if kj % 2 == 1:
            j = kj - 1
            pltpu.make_async_copy(table_hbm.at[idx_v.at[j]], bufs[j % 2],
                                  sems[j % 2]).wait()
            pltpu.sync_copy(bufs[j % 2],
                            out_hbm.at[pl.ds(base + j * _IDXW, _IDXW)])

    return sc_gather


_PACK = 4
_RC = 32768


def _repack_kernel(tt_ref, emap_ref, out_ref, *, vtotal):
    e = tt_ref.shape[0]
    lanes = _PACK * e
    sub = tt_ref.shape[1] // _PACK
    valid = vtotal - pl.program_id(0) * _RC
    col = lax.broadcasted_iota(jnp.int32, tt_ref.shape, 1)
    tt = jnp.where(col < valid, tt_ref[...], 0.0)
    acc = None
    for p in range(_PACK):
        d = lax.dot_general(
            tt[:, p * sub:(p + 1) * sub],
            emap_ref[:, p * lanes:(p + 1) * lanes],
            (((0,), (0,)), ((), ())),
            preferred_element_type=jnp.float32)
        acc = d if acc is None else acc + d
    out_ref[...] = acc


def _repack(table_t):
    e, v = table_t.shape
    lanes = _PACK * e
    sub = _RC // _PACK
    nblk = (v + _RC - 1) // _RC
    eye = jnp.eye(e, dtype=jnp.float32)
    emap = jnp.concatenate(
        [jnp.pad(eye, ((0, 0), (p * e, lanes - (p + 1) * e)))
         for p in range(_PACK)], axis=1)
    return pl.pallas_call(
        functools.partial(_repack_kernel, vtotal=v),
        grid=(nblk,),
        in_specs=[
            pl.BlockSpec((e, _RC), lambda k: (0, k)),
            pl.BlockSpec((e, _PACK * lanes), lambda k: (0, 0)),
        ],
        out_specs=pl.BlockSpec((sub, lanes), lambda k: (k, 0)),
        out_shape=jax.ShapeDtypeStruct((nblk * sub, lanes), jnp.float32),
        compiler_params=pltpu.CompilerParams(
            fuse_transposed_lhs_in_matmul=True,
        ),
    )(table_t, emap)


def _cell(x_t, h, c, wih_ref, whh_ref, bias_ref, h_dim):
    gates = (jnp.dot(x_t, wih_ref[...], preferred_element_type=jnp.float32)
             + jnp.dot(h, whh_ref[...], preferred_element_type=jnp.float32)
             + bias_ref[...])
    i_g = jax.nn.sigmoid(gates[:, 0:h_dim])
    f_g = jax.nn.sigmoid(gates[:, h_dim:2 * h_dim])
    g_g = jnp.tanh(gates[:, 2 * h_dim:3 * h_dim])
    o_g = jax.nn.sigmoid(gates[:, 3 * h_dim:4 * h_dim])
    c_new = f_g * c + i_g * g_g
    h_new = o_g * jnp.tanh(c_new)
    return h_new, c_new


def _x_sel(emb_ref, sel_ref):
    lanes = emb_ref.shape[1]
    sel_col = sel_ref[0].T
    grp = lax.broadcasted_iota(jnp.int32, emb_ref.shape, 1) // (lanes // _PACK)
    return jnp.where(grp == sel_col, emb_ref[...], 0.0)


def _lstm_first_kernel(emb_ref, sel_ref, wih_ref, whh_ref, bias_ref,
                       ho_ref, co_ref, h_ref, c_ref):
    t = pl.program_id(0)
    nt = pl.num_programs(0)
    h_dim = whh_ref.shape[0]

    @pl.when(t == 0)
    def _():
        h_ref[...] = jnp.zeros_like(h_ref)
        c_ref[...] = jnp.zeros_like(c_ref)

    x_t = _x_sel(emb_ref, sel_ref)
    h_new, c_new = _cell(x_t, h_ref[...], c_ref[...], wih_ref, whh_ref,
                         bias_ref, h_dim)
    h_ref[...] = h_new
    c_ref[...] = c_new

    @pl.when(t == nt - 1)
    def _():
        ho_ref[...] = h_new
        co_ref[...] = c_new


def _lstm_final_kernel(emb_ref, sel_ref, wih_ref, whh_ref, bias_ref,
                       h0_ref, c0_ref, wfc_ref, bfc_ref, out_ref,
                       h_ref, c_ref):
    t = pl.program_id(0)
    nt = pl.num_programs(0)
    h_dim = whh_ref.shape[0]

    @pl.when(t == 0)
    def _():
        h_ref[...] = h0_ref[...]
        c_ref[...] = c0_ref[...]

    x_t = _x_sel(emb_ref, sel_ref)
    h_new, c_new = _cell(x_t, h_ref[...], c_ref[...], wih_ref, whh_ref,
                         bias_ref, h_dim)
    h_ref[...] = h_new
    c_ref[...] = c_new

    @pl.when(t == nt - 1)
    def _():
        out_ref[...] = (jnp.dot(h_new, wfc_ref[...],
                                preferred_element_type=jnp.float32)
                        + bfc_ref[...])


def _lstm_first(emb_flat, sel, wih_t, whh_t, bias, b, t_steps):
    e = emb_flat.shape[1]
    h4 = wih_t.shape[1]
    h_dim = h4 // 4
    return pl.pallas_call(
        _lstm_first_kernel,
        grid=(t_steps,),
        in_specs=[
            pl.BlockSpec((b, e), lambda t: (t, 0)),
            pl.BlockSpec((1, 1, b), lambda t: (t, 0, 0)),
            pl.BlockSpec((e, h4), lambda t: (0, 0)),
            pl.BlockSpec((h_dim, h4), lambda t: (0, 0)),
            pl.BlockSpec((1, h4), lambda t: (0, 0)),
        ],
        out_specs=[
            pl.BlockSpec((b, h_dim), lambda t: (0, 0)),
            pl.BlockSpec((b, h_dim), lambda t: (0, 0)),
        ],
        out_shape=[
            jax.ShapeDtypeStruct((b, h_dim), jnp.float32),
            jax.ShapeDtypeStruct((b, h_dim), jnp.float32),
        ],
        scratch_shapes=[
            pltpu.VMEM((b, h_dim), jnp.float32),
            pltpu.VMEM((b, h_dim), jnp.float32),
        ],
        compiler_params=pltpu.CompilerParams(
            dimension_semantics=("arbitrary",),
        ),
    )(emb_flat, sel, wih_t, whh_t, bias)


def _lstm_final(emb_flat, sel, wih_t, whh_t, bias, h0, c0, wfc_t, bfc, b,
                t_steps):
    e = emb_flat.shape[1]
    h4 = wih_t.shape[1]
    h_dim = h4 // 4
    out_dim = wfc_t.shape[1]
    return pl.pallas_call(
        _lstm_final_kernel,
        grid=(t_steps,),
        in_specs=[
            pl.BlockSpec((b, e), lambda t: (t, 0)),
            pl.BlockSpec((1, 1, b), lambda t: (t, 0, 0)),
            pl.BlockSpec((e, h4), lambda t: (0, 0)),
            pl.BlockSpec((h_dim, h4), lambda t: (0, 0)),
            pl.BlockSpec((1, h4), lambda t: (0, 0)),
            pl.BlockSpec((b, h_dim), lambda t: (0, 0)),
            pl.BlockSpec((b, h_dim), lambda t: (0, 0)),
            pl.BlockSpec((h_dim, out_dim), lambda t: (0, 0)),
            pl.BlockSpec((1, out_dim), lambda t: (0, 0)),
        ],
        out_specs=pl.BlockSpec((b, out_dim), lambda t: (0, 0)),
        out_shape=jax.ShapeDtypeStruct((b, out_dim), jnp.float32),
        scratch_shapes=[
            pltpu.VMEM((b, h_dim), jnp.float32),
            pltpu.VMEM((b, h_dim), jnp.float32),
        ],
        compiler_params=pltpu.CompilerParams(
            dimension_semantics=("arbitrary",),
        ),
    )(emb_flat, sel, wih_t, whh_t, bias, h0, c0, wfc_t, bfc)


def kernel(x, table, W_ih, W_hh, b_ih, b_hh, W_fc, b_fc):
    b, t_steps = x.shape
    e = table.shape[1]
    h4 = W_ih.shape[0]
    out_dim = W_fc.shape[0]

    tbl_packed = _repack(table.T)
    xt = x.astype(jnp.int32).T
    sub = _RC // _PACK
    rows = (xt // _RC) * sub + (xt % sub)
    sels = (xt // sub) % _PACK
    th = t_steps // 2
    gat = _make_sc_gather(b * th, e * _PACK)
    idx_a = rows[:th].reshape(_NW, -1, _IDXW)
    idx_b = rows[th:].reshape(_NW, -1, _IDXW)
    emb_a = gat(idx_a, tbl_packed)
    emb_b = gat(idx_b, tbl_packed)
    sel_a = sels[:th].reshape(th, 1, b)
    sel_b = sels[th:].reshape(t_steps - th, 1, b)

    wih_t = jnp.tile(W_ih.T, (_PACK, 1))
    whh_t = W_hh.T
    bias = (b_ih + b_hh).reshape(1, h4)
    wfc_t = W_fc.T
    bfc = b_fc.reshape(1, out_dim)
    h0, c0 = _lstm_first(emb_a, sel_a, wih_t, whh_t, bias, b, th)
    return _lstm_final(emb_b, sel_b, wih_t, whh_t, bias, h0, c0, wfc_t, bfc,
                       b, t_steps - th)
